# baseline (speedup 1.0000x reference)
import functools

import jax
import jax.numpy as jnp
from jax import lax
from jax.experimental import pallas as pl
from jax.experimental.pallas import tpu as pltpu
from jax.experimental.pallas import tpu_sc as plsc

N_ROWS = 1000000
D = 64
B = 16384
Y_HIGH = 5.5

NC = 2
NS = 16
L = 16
NW = NC * NS
BPW = B // NW
CHUNK = 128
NCHUNK = BPW // CHUNK
HALF = BPW // 2


def _dots_body(u2_idx_hbm, m2_idx_hbm, uh_hbm, mh_hbm, uf_hbm, mf_hbm,
               dots_hbm, u_idx_v, m_idx_v, uh_v, mh_v, u_rows, m_rows,
               dots_v, sem):
    wid = lax.axis_index("s") * NC + lax.axis_index("c")
    base = pl.multiple_of(wid * BPW, BPW)

    pltpu.sync_copy(u2_idx_hbm.at[wid], u_idx_v)
    pltpu.sync_copy(m2_idx_hbm.at[wid], m_idx_v)
    pltpu.sync_copy(uh_hbm.at[wid], uh_v)
    pltpu.sync_copy(mh_hbm.at[wid], mh_v)

    iota = lax.iota(jnp.int32, L)

    for half in range(2):
        cps = []
        for j in range(2):
            ch = half * 2 + j
            dst = pl.ds(j * CHUNK, CHUNK)
            cps.append(pltpu.async_copy(
                uf_hbm.at[u_idx_v.at[ch]], u_rows.at[dst], sem))
            cps.append(pltpu.async_copy(
                mf_hbm.at[m_idx_v.at[ch]], m_rows.at[dst], sem))
        for cp in cps:
            cp.wait()

        def chunk(c, carry):
            c16 = pl.multiple_of(c * L, L)
            u16h = uh_v[0, pl.ds(half * HALF + c16, L)]
            m16h = mh_v[0, pl.ds(half * HALF + c16, L)]
            acc = jnp.zeros((L,), jnp.float32)
            for k in range(L):
                p = c16 + k
                uo = pl.multiple_of(u16h[k] * D, D)
                mo = pl.multiple_of(m16h[k] * D, D)
                p0 = (u_rows[p, pl.ds(uo + 0, 16)] *
                      m_rows[p, pl.ds(mo + 0, 16)])
                p1 = (u_rows[p, pl.ds(uo + 16, 16)] *
                      m_rows[p, pl.ds(mo + 16, 16)])
                p2 = (u_rows[p, pl.ds(uo + 32, 16)] *
                      m_rows[p, pl.ds(mo + 32, 16)])
                p3 = (u_rows[p, pl.ds(uo + 48, 16)] *
                      m_rows[p, pl.ds(mo + 48, 16)])
                s = jnp.sum((p0 + p1) + (p2 + p3))
                acc = jnp.where(iota == k, s, acc)
            dots_v[pl.ds(half * HALF + c16, L)] = acc
            return carry

        lax.fori_loop(0, HALF // L, chunk, 0)

    pltpu.sync_copy(dots_v, dots_hbm.at[pl.ds(base, BPW)])


def _bias_body(u_idx_hbm, m_idx_hbm, dots_hbm, ub_hbm, mb_hbm, out_hbm,
               u_idx_v, m_idx_v, u_bias_v, m_bias_v, dots_v, out_v, sem):
    wid = lax.axis_index("s") * NC + lax.axis_index("c")
    base = pl.multiple_of(wid * BPW, BPW)

    pltpu.sync_copy(u_idx_hbm.at[wid], u_idx_v)
    pltpu.sync_copy(m_idx_hbm.at[wid], m_idx_v)
    pltpu.sync_copy(dots_hbm.at[pl.ds(base, BPW)], dots_v)

    cps = []
    for j in range(NCHUNK):
        dst = pl.ds(j * CHUNK, CHUNK)
        cps.append(pltpu.async_copy(
            ub_hbm.at[u_idx_v.at[j]], u_bias_v.at[dst], sem))
        cps.append(pltpu.async_copy(
            mb_hbm.at[m_idx_v.at[j]], m_bias_v.at[dst], sem))
    for cp in cps:
        cp.wait()

    def chunk(c, carry):
        c16 = pl.multiple_of(c * L, L)
        t = (dots_v[pl.ds(c16, L)] + u_bias_v[pl.ds(c16, L)] +
             m_bias_v[pl.ds(c16, L)])
        out_v[pl.ds(c16, L)] = Y_HIGH / (1.0 + jnp.exp(-t))
        return carry

    lax.fori_loop(0, BPW // L, chunk, 0)

    pltpu.sync_copy(out_v, out_hbm.at[pl.ds(base, BPW)])


_TRW = 8192
_TRH = _TRW // 2
_TRG = (N_ROWS + _TRW - 1) // _TRW
K_ROWS = _TRG * _TRH


def _tr_body(x_ref, o_ref):
    ident = jnp.eye(D, dtype=jnp.float32)
    dims = (((0,), (0,)), ((), ()))
    o_ref[:, 0:D] = lax.dot_general(
        x_ref[:, 0:_TRH], ident, dims,
        precision=lax.Precision.HIGHEST,
        preferred_element_type=jnp.float32)
    o_ref[:, D:2 * D] = lax.dot_general(
        x_ref[:, _TRH:_TRW], ident, dims,
        precision=lax.Precision.HIGHEST,
        preferred_element_type=jnp.float32)


def _pair_table(table_t):
    return pl.pallas_call(
        _tr_body,
        grid=(_TRG,),
        in_specs=[pl.BlockSpec((D, _TRW), lambda i: (0, i))],
        out_specs=pl.BlockSpec((_TRH, 2 * D), lambda i: (i, 0)),
        out_shape=jax.ShapeDtypeStruct((K_ROWS, 2 * D), jnp.float32),
    )(table_t)


@jax.jit
def _run(u2_idx, m2_idx, uh, mh, u_idx, m_idx, uf2, user_bias, mf2,
         movie_bias):
    uf2 = _pair_table(uf2)
    mf2 = _pair_table(mf2)
    mesh = plsc.VectorSubcoreMesh(core_axis_name="c", subcore_axis_name="s")
    dots_f = pl.kernel(
        _dots_body,
        mesh=mesh,
        compiler_params=pltpu.CompilerParams(needs_layout_passes=False),
        out_type=jax.ShapeDtypeStruct((B,), jnp.float32),
        scratch_types=[
            pltpu.VMEM((NCHUNK, CHUNK), jnp.int32),
            pltpu.VMEM((NCHUNK, CHUNK), jnp.int32),
            pltpu.VMEM((1, BPW), jnp.int32),
            pltpu.VMEM((1, BPW), jnp.int32),
            pltpu.VMEM((HALF, 2 * D), jnp.float32),
            pltpu.VMEM((HALF, 2 * D), jnp.float32),
            pltpu.VMEM((BPW,), jnp.float32),
            pltpu.SemaphoreType.DMA,
        ],
    )
    dots = dots_f(u2_idx, m2_idx, uh, mh, uf2, mf2)

    bias_f = pl.kernel(
        _bias_body,
        mesh=mesh,
        compiler_params=pltpu.CompilerParams(
            needs_layout_passes=False, use_tc_tiling_on_sc=False),
        out_type=jax.ShapeDtypeStruct((B,), jnp.float32),
        scratch_types=[
            pltpu.VMEM((NCHUNK, CHUNK), jnp.int32),
            pltpu.VMEM((NCHUNK, CHUNK), jnp.int32),
            pltpu.VMEM((BPW,), jnp.float32),
            pltpu.VMEM((BPW,), jnp.float32),
            pltpu.VMEM((BPW,), jnp.float32),
            pltpu.VMEM((BPW,), jnp.float32),
            pltpu.SemaphoreType.DMA,
        ],
    )
    return bias_f(u_idx, m_idx, dots, user_bias, movie_bias)


def kernel(x, user_factors, user_bias, movie_factors, movie_bias):
    xu = x[:, 0]
    xm = x[:, 1]
    u2_idx = ((xu // _TRW) * _TRH + xu % _TRH).reshape(NW, NCHUNK, CHUNK)
    m2_idx = ((xm // _TRW) * _TRH + xm % _TRH).reshape(NW, NCHUNK, CHUNK)
    uh = ((xu % _TRW) // _TRH).reshape(NW, 1, BPW)
    mh = ((xm % _TRW) // _TRH).reshape(NW, 1, BPW)
    u_idx = xu.reshape(NW, NCHUNK, CHUNK)
    m_idx = xm.reshape(NW, NCHUNK, CHUNK)
    uf2 = user_factors.T
    mf2 = movie_factors.T
    out = _run(u2_idx, m2_idx, uh, mh, u_idx, m_idx, uf2,
               user_bias.reshape(-1), mf2, movie_bias.reshape(-1))
    return out.reshape(B, 1)


# speedup vs baseline: 1.7702x; 1.7702x over previous
import functools

import jax
import jax.numpy as jnp
from jax import lax
from jax.experimental import pallas as pl
from jax.experimental.pallas import tpu as pltpu
from jax.experimental.pallas import tpu_sc as plsc

N_ROWS = 1000000
D = 64
B = 16384
Y_HIGH = 5.5

NC = 2
NS = 16
L = 16
NW = NC * NS
BPW = B // NW
CHUNK = 128
NCHUNK = BPW // CHUNK
HALF = BPW // 2


def _dots_body(u2_idx_hbm, m2_idx_hbm, uh_hbm, mh_hbm, uf_hbm, mf_hbm,
               dots_hbm, u_idx_v, m_idx_v, uh_v, mh_v, u_rows, m_rows,
               dots_v, sem):
    wid = lax.axis_index("s") * NC + lax.axis_index("c")
    base = pl.multiple_of(wid * BPW, BPW)

    pltpu.sync_copy(u2_idx_hbm.at[wid], u_idx_v)
    pltpu.sync_copy(m2_idx_hbm.at[wid], m_idx_v)
    pltpu.sync_copy(uh_hbm.at[wid], uh_v)
    pltpu.sync_copy(mh_hbm.at[wid], mh_v)

    iota = lax.iota(jnp.int32, L)

    for half in range(2):
        cps = []
        for j in range(2):
            ch = half * 2 + j
            dst = pl.ds(j * CHUNK, CHUNK)
            cps.append(pltpu.async_copy(
                uf_hbm.at[u_idx_v.at[ch]], u_rows.at[dst], sem))
            cps.append(pltpu.async_copy(
                mf_hbm.at[m_idx_v.at[ch]], m_rows.at[dst], sem))
        for cp in cps:
            cp.wait()

        def chunk(c, carry):
            c16 = pl.multiple_of(c * L, L)
            u16h = uh_v[0, pl.ds(half * HALF + c16, L)]
            m16h = mh_v[0, pl.ds(half * HALF + c16, L)]
            acc = jnp.zeros((L,), jnp.float32)
            for k in range(L):
                p = c16 + k
                uo = pl.multiple_of(u16h[k] * D, D)
                mo = pl.multiple_of(m16h[k] * D, D)
                p0 = (u_rows[p, pl.ds(uo + 0, 16)] *
                      m_rows[p, pl.ds(mo + 0, 16)])
                p1 = (u_rows[p, pl.ds(uo + 16, 16)] *
                      m_rows[p, pl.ds(mo + 16, 16)])
                p2 = (u_rows[p, pl.ds(uo + 32, 16)] *
                      m_rows[p, pl.ds(mo + 32, 16)])
                p3 = (u_rows[p, pl.ds(uo + 48, 16)] *
                      m_rows[p, pl.ds(mo + 48, 16)])
                s = jnp.sum((p0 + p1) + (p2 + p3))
                acc = jnp.where(iota == k, s, acc)
            dots_v[pl.ds(half * HALF + c16, L)] = acc
            return carry

        lax.fori_loop(0, HALF // L, chunk, 0)

    pltpu.sync_copy(dots_v, dots_hbm.at[pl.ds(base, BPW)])


def _bias_body(u_idx_hbm, m_idx_hbm, dots_hbm, ub_hbm, mb_hbm, out_hbm,
               u_idx_v, m_idx_v, u_bias_v, m_bias_v, dots_v, out_v, sem):
    wid = lax.axis_index("s") * NC + lax.axis_index("c")
    base = pl.multiple_of(wid * BPW, BPW)

    pltpu.sync_copy(u_idx_hbm.at[wid], u_idx_v)
    pltpu.sync_copy(m_idx_hbm.at[wid], m_idx_v)
    pltpu.sync_copy(dots_hbm.at[pl.ds(base, BPW)], dots_v)

    cps = []
    for j in range(NCHUNK):
        dst = pl.ds(j * CHUNK, CHUNK)
        cps.append(pltpu.async_copy(
            ub_hbm.at[u_idx_v.at[j]], u_bias_v.at[dst], sem))
        cps.append(pltpu.async_copy(
            mb_hbm.at[m_idx_v.at[j]], m_bias_v.at[dst], sem))
    for cp in cps:
        cp.wait()

    def chunk(c, carry):
        c16 = pl.multiple_of(c * L, L)
        t = (dots_v[pl.ds(c16, L)] + u_bias_v[pl.ds(c16, L)] +
             m_bias_v[pl.ds(c16, L)])
        out_v[pl.ds(c16, L)] = Y_HIGH / (1.0 + jnp.exp(-t))
        return carry

    lax.fori_loop(0, BPW // L, chunk, 0)

    pltpu.sync_copy(out_v, out_hbm.at[pl.ds(base, BPW)])


_TRW = 8192
_TRH = _TRW // 2
_TRG = (N_ROWS + _TRW - 1) // _TRW
K_ROWS = _TRG * _TRH


def _tr_body(x_ref, o_ref):
    ident = jnp.eye(D, dtype=jnp.float32)
    dims = (((0,), (0,)), ((), ()))
    o_ref[:, 0:D] = lax.dot_general(
        x_ref[:, 0:_TRH], ident, dims,
        preferred_element_type=jnp.float32)
    o_ref[:, D:2 * D] = lax.dot_general(
        x_ref[:, _TRH:_TRW], ident, dims,
        preferred_element_type=jnp.float32)


def _pair_table(table_t):
    return pl.pallas_call(
        _tr_body,
        grid=(_TRG,),
        in_specs=[pl.BlockSpec((D, _TRW), lambda i: (0, i))],
        out_specs=pl.BlockSpec((_TRH, 2 * D), lambda i: (i, 0)),
        out_shape=jax.ShapeDtypeStruct((K_ROWS, 2 * D), jnp.float32),
    )(table_t)


@jax.jit
def _run(u2_idx, m2_idx, uh, mh, u_idx, m_idx, uf2, user_bias, mf2,
         movie_bias):
    uf2 = _pair_table(uf2)
    mf2 = _pair_table(mf2)
    mesh = plsc.VectorSubcoreMesh(core_axis_name="c", subcore_axis_name="s")
    dots_f = pl.kernel(
        _dots_body,
        mesh=mesh,
        compiler_params=pltpu.CompilerParams(needs_layout_passes=False),
        out_type=jax.ShapeDtypeStruct((B,), jnp.float32),
        scratch_types=[
            pltpu.VMEM((NCHUNK, CHUNK), jnp.int32),
            pltpu.VMEM((NCHUNK, CHUNK), jnp.int32),
            pltpu.VMEM((1, BPW), jnp.int32),
            pltpu.VMEM((1, BPW), jnp.int32),
            pltpu.VMEM((HALF, 2 * D), jnp.float32),
            pltpu.VMEM((HALF, 2 * D), jnp.float32),
            pltpu.VMEM((BPW,), jnp.float32),
            pltpu.SemaphoreType.DMA,
        ],
    )
    dots = dots_f(u2_idx, m2_idx, uh, mh, uf2, mf2)

    bias_f = pl.kernel(
        _bias_body,
        mesh=mesh,
        compiler_params=pltpu.CompilerParams(
            needs_layout_passes=False, use_tc_tiling_on_sc=False),
        out_type=jax.ShapeDtypeStruct((B,), jnp.float32),
        scratch_types=[
            pltpu.VMEM((NCHUNK, CHUNK), jnp.int32),
            pltpu.VMEM((NCHUNK, CHUNK), jnp.int32),
            pltpu.VMEM((BPW,), jnp.float32),
            pltpu.VMEM((BPW,), jnp.float32),
            pltpu.VMEM((BPW,), jnp.float32),
            pltpu.VMEM((BPW,), jnp.float32),
            pltpu.SemaphoreType.DMA,
        ],
    )
    return bias_f(u_idx, m_idx, dots, user_bias, movie_bias)


def kernel(x, user_factors, user_bias, movie_factors, movie_bias):
    xu = x[:, 0]
    xm = x[:, 1]
    u2_idx = ((xu // _TRW) * _TRH + xu % _TRH).reshape(NW, NCHUNK, CHUNK)
    m2_idx = ((xm // _TRW) * _TRH + xm % _TRH).reshape(NW, NCHUNK, CHUNK)
    uh = ((xu % _TRW) // _TRH).reshape(NW, 1, BPW)
    mh = ((xm % _TRW) // _TRH).reshape(NW, 1, BPW)
    u_idx = xu.reshape(NW, NCHUNK, CHUNK)
    m_idx = xm.reshape(NW, NCHUNK, CHUNK)
    uf2 = user_factors.T
    mf2 = movie_factors.T
    out = _run(u2_idx, m2_idx, uh, mh, u_idx, m_idx, uf2,
               user_bias.reshape(-1), mf2, movie_bias.reshape(-1))
    return out.reshape(B, 1)


# speedup vs baseline: 1.9769x; 1.1168x over previous
import functools

import jax
import jax.numpy as jnp
from jax import lax
from jax.experimental import pallas as pl
from jax.experimental.pallas import tpu as pltpu
from jax.experimental.pallas import tpu_sc as plsc

N_ROWS = 1000000
D = 64
B = 16384
Y_HIGH = 5.5

NC = 2
NS = 16
L = 16
NW = NC * NS
BPW = B // NW
CHUNK = 128
NCHUNK = BPW // CHUNK
HALF = BPW // 2


def _dots_body(u2_idx_hbm, m2_idx_hbm, uh_hbm, mh_hbm, uf_hbm, mf_hbm,
               dots_hbm, u_idx_v, m_idx_v, uh_v, mh_v, u_rows, m_rows,
               dots_v, sem):
    wid = lax.axis_index("s") * NC + lax.axis_index("c")
    base = pl.multiple_of(wid * BPW, BPW)

    pltpu.sync_copy(u2_idx_hbm.at[wid], u_idx_v)
    pltpu.sync_copy(m2_idx_hbm.at[wid], m_idx_v)
    pltpu.sync_copy(uh_hbm.at[wid], uh_v)
    pltpu.sync_copy(mh_hbm.at[wid], mh_v)

    iota = lax.iota(jnp.int32, L)

    for half in range(2):
        cps = []
        for j in range(2):
            ch = half * 2 + j
            dst = pl.ds(j * CHUNK, CHUNK)
            cps.append(pltpu.async_copy(
                uf_hbm.at[u_idx_v.at[ch]], u_rows.at[dst], sem))
            cps.append(pltpu.async_copy(
                mf_hbm.at[m_idx_v.at[ch]], m_rows.at[dst], sem))
        for cp in cps:
            cp.wait()

        def chunk(c, carry):
            c16 = pl.multiple_of(c * L, L)
            u16h = uh_v[0, pl.ds(half * HALF + c16, L)]
            m16h = mh_v[0, pl.ds(half * HALF + c16, L)]
            acc = jnp.zeros((L,), jnp.float32)
            for k in range(L):
                p = c16 + k
                uo = pl.multiple_of(u16h[k] * D, D)
                mo = pl.multiple_of(m16h[k] * D, D)
                p0 = (u_rows[p, pl.ds(uo + 0, 16)] *
                      m_rows[p, pl.ds(mo + 0, 16)])
                p1 = (u_rows[p, pl.ds(uo + 16, 16)] *
                      m_rows[p, pl.ds(mo + 16, 16)])
                p2 = (u_rows[p, pl.ds(uo + 32, 16)] *
                      m_rows[p, pl.ds(mo + 32, 16)])
                p3 = (u_rows[p, pl.ds(uo + 48, 16)] *
                      m_rows[p, pl.ds(mo + 48, 16)])
                s = jnp.sum((p0 + p1) + (p2 + p3))
                acc = jnp.where(iota == k, s, acc)
            dots_v[pl.ds(half * HALF + c16, L)] = acc
            return carry

        lax.fori_loop(0, HALF // L, chunk, 0)

    pltpu.sync_copy(dots_v, dots_hbm.at[pl.ds(base, BPW)])


def _bias_body(u_idx_hbm, m_idx_hbm, dots_hbm, ub_hbm, mb_hbm, out_hbm,
               u_idx_v, m_idx_v, u_bias_v, m_bias_v, dots_v, out_v, sem):
    wid = lax.axis_index("s") * NC + lax.axis_index("c")
    base = pl.multiple_of(wid * BPW, BPW)

    pltpu.sync_copy(u_idx_hbm.at[wid], u_idx_v)
    pltpu.sync_copy(m_idx_hbm.at[wid], m_idx_v)
    pltpu.sync_copy(dots_hbm.at[pl.ds(base, BPW)], dots_v)

    cps = []
    for j in range(NCHUNK):
        dst = pl.ds(j * CHUNK, CHUNK)
        cps.append(pltpu.async_copy(
            ub_hbm.at[u_idx_v.at[j]], u_bias_v.at[dst], sem))
        cps.append(pltpu.async_copy(
            mb_hbm.at[m_idx_v.at[j]], m_bias_v.at[dst], sem))
    for cp in cps:
        cp.wait()

    def chunk(c, carry):
        c16 = pl.multiple_of(c * L, L)
        t = (dots_v[pl.ds(c16, L)] + u_bias_v[pl.ds(c16, L)] +
             m_bias_v[pl.ds(c16, L)])
        out_v[pl.ds(c16, L)] = Y_HIGH / (1.0 + jnp.exp(-t))
        return carry

    lax.fori_loop(0, BPW // L, chunk, 0)

    pltpu.sync_copy(out_v, out_hbm.at[pl.ds(base, BPW)])


_TRW = 16384
_TRH = _TRW // 2
_TRG = (N_ROWS + _TRW - 1) // _TRW
K_ROWS = _TRG * _TRH


def _tr_body(x_ref, o_ref):
    ident = jnp.eye(D, dtype=jnp.float32)
    dims = (((0,), (0,)), ((), ()))
    o_ref[:, 0:D] = lax.dot_general(
        x_ref[:, 0:_TRH], ident, dims,
        preferred_element_type=jnp.float32)
    o_ref[:, D:2 * D] = lax.dot_general(
        x_ref[:, _TRH:_TRW], ident, dims,
        preferred_element_type=jnp.float32)


def _pair_table(table_t):
    return pl.pallas_call(
        _tr_body,
        grid=(_TRG,),
        in_specs=[pl.BlockSpec((D, _TRW), lambda i: (0, i))],
        out_specs=pl.BlockSpec((_TRH, 2 * D), lambda i: (i, 0)),
        out_shape=jax.ShapeDtypeStruct((K_ROWS, 2 * D), jnp.float32),
    )(table_t)


@jax.jit
def _run(u2_idx, m2_idx, uh, mh, u_idx, m_idx, uf2, user_bias, mf2,
         movie_bias):
    uf2 = _pair_table(uf2)
    mf2 = _pair_table(mf2)
    mesh = plsc.VectorSubcoreMesh(core_axis_name="c", subcore_axis_name="s")
    dots_f = pl.kernel(
        _dots_body,
        mesh=mesh,
        compiler_params=pltpu.CompilerParams(needs_layout_passes=False),
        out_type=jax.ShapeDtypeStruct((B,), jnp.float32),
        scratch_types=[
            pltpu.VMEM((NCHUNK, CHUNK), jnp.int32),
            pltpu.VMEM((NCHUNK, CHUNK), jnp.int32),
            pltpu.VMEM((1, BPW), jnp.int32),
            pltpu.VMEM((1, BPW), jnp.int32),
            pltpu.VMEM((HALF, 2 * D), jnp.float32),
            pltpu.VMEM((HALF, 2 * D), jnp.float32),
            pltpu.VMEM((BPW,), jnp.float32),
            pltpu.SemaphoreType.DMA,
        ],
    )
    dots = dots_f(u2_idx, m2_idx, uh, mh, uf2, mf2)

    bias_f = pl.kernel(
        _bias_body,
        mesh=mesh,
        compiler_params=pltpu.CompilerParams(
            needs_layout_passes=False, use_tc_tiling_on_sc=False),
        out_type=jax.ShapeDtypeStruct((B,), jnp.float32),
        scratch_types=[
            pltpu.VMEM((NCHUNK, CHUNK), jnp.int32),
            pltpu.VMEM((NCHUNK, CHUNK), jnp.int32),
            pltpu.VMEM((BPW,), jnp.float32),
            pltpu.VMEM((BPW,), jnp.float32),
            pltpu.VMEM((BPW,), jnp.float32),
            pltpu.VMEM((BPW,), jnp.float32),
            pltpu.SemaphoreType.DMA,
        ],
    )
    return bias_f(u_idx, m_idx, dots, user_bias, movie_bias)


def kernel(x, user_factors, user_bias, movie_factors, movie_bias):
    xu = x[:, 0]
    xm = x[:, 1]
    u2_idx = ((xu // _TRW) * _TRH + xu % _TRH).reshape(NW, NCHUNK, CHUNK)
    m2_idx = ((xm // _TRW) * _TRH + xm % _TRH).reshape(NW, NCHUNK, CHUNK)
    uh = ((xu % _TRW) // _TRH).reshape(NW, 1, BPW)
    mh = ((xm % _TRW) // _TRH).reshape(NW, 1, BPW)
    u_idx = xu.reshape(NW, NCHUNK, CHUNK)
    m_idx = xm.reshape(NW, NCHUNK, CHUNK)
    uf2 = user_factors.T
    mf2 = movie_factors.T
    out = _run(u2_idx, m2_idx, uh, mh, u_idx, m_idx, uf2,
               user_bias.reshape(-1), mf2, movie_bias.reshape(-1))
    return out.reshape(B, 1)


# speedup vs baseline: 2.0846x; 1.0545x over previous
import functools

import jax
import jax.numpy as jnp
from jax import lax
from jax.experimental import pallas as pl
from jax.experimental.pallas import tpu as pltpu
from jax.experimental.pallas import tpu_sc as plsc

N_ROWS = 1000000
D = 64
B = 16384
Y_HIGH = 5.5

NC = 2
NS = 16
L = 16
NW = NC * NS
BPW = B // NW
CHUNK = 128
NCHUNK = BPW // CHUNK
HALF = BPW // 2


def _dots_body(u2_idx_hbm, m2_idx_hbm, uh_hbm, mh_hbm, uf_hbm, mf_hbm,
               dots_hbm, u_idx_v, m_idx_v, uh_v, mh_v, u_rows, m_rows,
               dots_v, sem):
    wid = lax.axis_index("s") * NC + lax.axis_index("c")
    base = pl.multiple_of(wid * BPW, BPW)

    pltpu.sync_copy(u2_idx_hbm.at[wid], u_idx_v)
    pltpu.sync_copy(m2_idx_hbm.at[wid], m_idx_v)
    pltpu.sync_copy(uh_hbm.at[wid], uh_v)
    pltpu.sync_copy(mh_hbm.at[wid], mh_v)

    iota = lax.iota(jnp.int32, L)

    for half in range(2):
        cps = []
        for j in range(2):
            ch = half * 2 + j
            dst = pl.ds(j * CHUNK, CHUNK)
            cps.append(pltpu.async_copy(
                uf_hbm.at[u_idx_v.at[ch]], u_rows.at[dst], sem))
            cps.append(pltpu.async_copy(
                mf_hbm.at[m_idx_v.at[ch]], m_rows.at[dst], sem))
        for cp in cps:
            cp.wait()

        def chunk(c, carry):
            c16 = pl.multiple_of(c * L, L)
            u16h = uh_v[0, pl.ds(half * HALF + c16, L)]
            m16h = mh_v[0, pl.ds(half * HALF + c16, L)]
            acc = jnp.zeros((L,), jnp.float32)
            for k in range(L):
                p = c16 + k
                uo = pl.multiple_of(u16h[k] * D, D)
                mo = pl.multiple_of(m16h[k] * D, D)
                p0 = (u_rows[p, pl.ds(uo + 0, 16)] *
                      m_rows[p, pl.ds(mo + 0, 16)])
                p1 = (u_rows[p, pl.ds(uo + 16, 16)] *
                      m_rows[p, pl.ds(mo + 16, 16)])
                p2 = (u_rows[p, pl.ds(uo + 32, 16)] *
                      m_rows[p, pl.ds(mo + 32, 16)])
                p3 = (u_rows[p, pl.ds(uo + 48, 16)] *
                      m_rows[p, pl.ds(mo + 48, 16)])
                s = jnp.sum((p0 + p1) + (p2 + p3))
                acc = jnp.where(iota == k, s, acc)
            dots_v[pl.ds(half * HALF + c16, L)] = acc
            return carry

        lax.fori_loop(0, HALF // L, chunk, 0)

    pltpu.sync_copy(dots_v, dots_hbm.at[pl.ds(base, BPW)])


def _bias_body(u_idx_hbm, m_idx_hbm, dots_hbm, ub_hbm, mb_hbm, out_hbm,
               u_idx_v, m_idx_v, u_bias_v, m_bias_v, dots_v, out_v, sem):
    wid = lax.axis_index("s") * NC + lax.axis_index("c")
    base = pl.multiple_of(wid * BPW, BPW)

    pltpu.sync_copy(u_idx_hbm.at[wid], u_idx_v)
    pltpu.sync_copy(m_idx_hbm.at[wid], m_idx_v)
    pltpu.sync_copy(dots_hbm.at[pl.ds(base, BPW)], dots_v)

    cps = []
    for j in range(NCHUNK):
        dst = pl.ds(j * CHUNK, CHUNK)
        cps.append(pltpu.async_copy(
            ub_hbm.at[u_idx_v.at[j]], u_bias_v.at[dst], sem))
        cps.append(pltpu.async_copy(
            mb_hbm.at[m_idx_v.at[j]], m_bias_v.at[dst], sem))
    for cp in cps:
        cp.wait()

    def chunk(c, carry):
        c16 = pl.multiple_of(c * L, L)
        t = (dots_v[pl.ds(c16, L)] + u_bias_v[pl.ds(c16, L)] +
             m_bias_v[pl.ds(c16, L)])
        out_v[pl.ds(c16, L)] = Y_HIGH / (1.0 + jnp.exp(-t))
        return carry

    lax.fori_loop(0, BPW // L, chunk, 0)

    pltpu.sync_copy(out_v, out_hbm.at[pl.ds(base, BPW)])


_TRW = 32768
_TRH = _TRW // 2
_TRG = (N_ROWS + _TRW - 1) // _TRW
K_ROWS = _TRG * _TRH


def _tr_body(x_ref, o_ref):
    ident = jnp.eye(D, dtype=jnp.float32)
    dims = (((0,), (0,)), ((), ()))
    o_ref[:, 0:D] = lax.dot_general(
        x_ref[:, 0:_TRH], ident, dims,
        preferred_element_type=jnp.float32)
    o_ref[:, D:2 * D] = lax.dot_general(
        x_ref[:, _TRH:_TRW], ident, dims,
        preferred_element_type=jnp.float32)


def _pair_table(table_t):
    return pl.pallas_call(
        _tr_body,
        grid=(_TRG,),
        in_specs=[pl.BlockSpec((D, _TRW), lambda i: (0, i))],
        out_specs=pl.BlockSpec((_TRH, 2 * D), lambda i: (i, 0)),
        out_shape=jax.ShapeDtypeStruct((K_ROWS, 2 * D), jnp.float32),
    )(table_t)


@jax.jit
def _run(u2_idx, m2_idx, uh, mh, u_idx, m_idx, uf2, user_bias, mf2,
         movie_bias):
    uf2 = _pair_table(uf2)
    mf2 = _pair_table(mf2)
    mesh = plsc.VectorSubcoreMesh(core_axis_name="c", subcore_axis_name="s")
    dots_f = pl.kernel(
        _dots_body,
        mesh=mesh,
        compiler_params=pltpu.CompilerParams(needs_layout_passes=False),
        out_type=jax.ShapeDtypeStruct((B,), jnp.float32),
        scratch_types=[
            pltpu.VMEM((NCHUNK, CHUNK), jnp.int32),
            pltpu.VMEM((NCHUNK, CHUNK), jnp.int32),
            pltpu.VMEM((1, BPW), jnp.int32),
            pltpu.VMEM((1, BPW), jnp.int32),
            pltpu.VMEM((HALF, 2 * D), jnp.float32),
            pltpu.VMEM((HALF, 2 * D), jnp.float32),
            pltpu.VMEM((BPW,), jnp.float32),
            pltpu.SemaphoreType.DMA,
        ],
    )
    dots = dots_f(u2_idx, m2_idx, uh, mh, uf2, mf2)

    bias_f = pl.kernel(
        _bias_body,
        mesh=mesh,
        compiler_params=pltpu.CompilerParams(
            needs_layout_passes=False, use_tc_tiling_on_sc=False),
        out_type=jax.ShapeDtypeStruct((B,), jnp.float32),
        scratch_types=[
            pltpu.VMEM((NCHUNK, CHUNK), jnp.int32),
            pltpu.VMEM((NCHUNK, CHUNK), jnp.int32),
            pltpu.VMEM((BPW,), jnp.float32),
            pltpu.VMEM((BPW,), jnp.float32),
            pltpu.VMEM((BPW,), jnp.float32),
            pltpu.VMEM((BPW,), jnp.float32),
            pltpu.SemaphoreType.DMA,
        ],
    )
    return bias_f(u_idx, m_idx, dots, user_bias, movie_bias)


def kernel(x, user_factors, user_bias, movie_factors, movie_bias):
    xu = x[:, 0]
    xm = x[:, 1]
    u2_idx = ((xu // _TRW) * _TRH + xu % _TRH).reshape(NW, NCHUNK, CHUNK)
    m2_idx = ((xm // _TRW) * _TRH + xm % _TRH).reshape(NW, NCHUNK, CHUNK)
    uh = ((xu % _TRW) // _TRH).reshape(NW, 1, BPW)
    mh = ((xm % _TRW) // _TRH).reshape(NW, 1, BPW)
    u_idx = xu.reshape(NW, NCHUNK, CHUNK)
    m_idx = xm.reshape(NW, NCHUNK, CHUNK)
    uf2 = user_factors.T
    mf2 = movie_factors.T
    out = _run(u2_idx, m2_idx, uh, mh, u_idx, m_idx, uf2,
               user_bias.reshape(-1), mf2, movie_bias.reshape(-1))
    return out.reshape(B, 1)
